# baseline (speedup 1.0000x reference)
import functools

import jax
import jax.numpy as jnp
from jax import lax
from jax.experimental import pallas as pl
from jax.experimental.pallas import tpu as pltpu
from jax.experimental.pallas import tpu_sc as plsc

VOCAB = 1000000
HIDDEN = 64
BATCH = 4096
SEQ = 200

NC = 2
NS = 16
NW = NC * NS

TOTAL = BATCH * SEQ
PER_W = TOTAL // NW
IW = 128
CHUNK = 512
NSUB = CHUNK // IW
NCHUNK = PER_W // CHUNK
IDX_ROWS = PER_W // IW


def _gather_kernel(emb_hbm, x_hbm, out_hbm, idx_v, rows0, rows1, gsem0,
                   gsem1, osem0, osem1):
  wid = lax.axis_index("s") * NC + lax.axis_index("c")
  base = wid * PER_W

  pltpu.sync_copy(x_hbm.at[pl.ds(wid * IDX_ROWS, IDX_ROWS)], idx_v)

  rows = (rows0, rows1)
  gsems = (gsem0, gsem1)
  osems = (osem0, osem1)

  def out_copy(buf, c, sem):
    return pltpu.make_async_copy(
        buf, out_hbm.at[pl.ds(base + c * CHUNK, CHUNK)], sem)

  def do_chunk(c, b, first):
    buf, gsem, osem = rows[b], gsems[b], osems[b]
    if not first:
      out_copy(buf, c - 2, osem).wait()
    cps = [
        pltpu.make_async_copy(
            emb_hbm.at[idx_v.at[c * NSUB + j]],
            buf.at[pl.ds(j * IW, IW)], gsem)
        for j in range(NSUB)
    ]
    for cp in cps:
      cp.start()
    for cp in cps:
      cp.wait()
    out_copy(buf, c, osem).start()

  do_chunk(0, 0, True)
  do_chunk(1, 1, True)

  def body(i):
    do_chunk(2 * i, 0, False)
    do_chunk(2 * i + 1, 1, False)

  pl.loop(1, NCHUNK // 2)(body)

  out_copy(rows[0], NCHUNK - 2, osems[0]).wait()
  out_copy(rows[1], NCHUNK - 1, osems[1]).wait()


@jax.jit
def kernel(x, emb):
  mesh = plsc.VectorSubcoreMesh(core_axis_name="c", subcore_axis_name="s")
  gather = functools.partial(
      pl.kernel,
      mesh=mesh,
      out_type=jax.ShapeDtypeStruct((TOTAL, HIDDEN), jnp.float32),
      scratch_types=[
          pltpu.VMEM((IDX_ROWS, IW), jnp.int32),
          pltpu.VMEM((CHUNK, HIDDEN), jnp.float32),
          pltpu.VMEM((CHUNK, HIDDEN), jnp.float32),
          pltpu.SemaphoreType.DMA,
          pltpu.SemaphoreType.DMA,
          pltpu.SemaphoreType.DMA,
          pltpu.SemaphoreType.DMA,
      ],
      compiler_params=pltpu.CompilerParams(use_tc_tiling_on_sc=False),
  )(_gather_kernel)
  xf = x.reshape(TOTAL // IW, IW).astype(jnp.int32)
  out = gather(emb, xf)
  return out.reshape(BATCH, SEQ, HIDDEN)
